# Initial kernel scaffold; baseline (speedup 1.0000x reference)
import jax
import jax.numpy as jnp
from jax.experimental import pallas as pl


def kernel(t_cls_0, t_cls_1, t_cls_2, t_bbox_0, t_bbox_1, t_bbox_2, t_ang_0, t_ang_1, t_ang_2, t_cent_0, t_cent_1, t_cent_2, s_cls_0, s_cls_1, s_cls_2, s_bbox_0, s_bbox_1, s_bbox_2, s_ang_0, s_ang_1, s_ang_2, s_cent_0, s_cent_1, s_cent_2):
    raise NotImplementedError("write your pallas kernel here")


# speedup vs baseline: 5.6458x; 5.6458x over previous
import jax
import jax.numpy as jnp
from jax import lax
from jax.experimental import pallas as pl

_CLS = 16
_SIZES = [128, 64, 32]
_B = 2
_K = 2000


def _bce(p, t):
    return -(t * jnp.clip(jnp.log(p), -100.0, None)
             + (1.0 - t) * jnp.clip(jnp.log(1.0 - p), -100.0, None))


def _smooth_l1(a, b):
    d = jnp.abs(a - b)
    return jnp.where(d < 1.0, 0.5 * d * d, d - 0.5)


def _topk_mask(mv, k):
    n = mv.shape[0] * mv.shape[1]
    u = lax.bitcast_convert_type(mv, jnp.int32)

    def vsearch(i, lohi):
        lo, hi = lohi
        mid = (lo + hi) // 2
        cnt = jnp.sum((u >= mid).astype(jnp.int32))
        good = cnt >= k
        return (jnp.where(good, mid, lo), jnp.where(good, hi, mid))

    lo, hi = lax.fori_loop(0, 31, vsearch, (jnp.int32(0), jnp.int32(1 << 30)))
    t_bits = lo
    cnt_gt = jnp.sum((u > t_bits).astype(jnp.int32))
    need = k - cnt_gt
    tie = (u == t_bits)

    idx = (lax.broadcasted_iota(jnp.int32, mv.shape, 0) * mv.shape[1]
           + lax.broadcasted_iota(jnp.int32, mv.shape, 1))

    def isearch(i, lohi):
        lo, hi = lohi
        mid = (lo + hi) // 2
        cnt = jnp.sum((tie & (idx < mid)).astype(jnp.int32))
        good = cnt >= need
        return (jnp.where(good, lo, mid), jnp.where(good, mid, hi))

    ilo, ihi = lax.fori_loop(0, 15, isearch, (jnp.int32(0), jnp.int32(n)))
    return (u > t_bits) | (tie & (idx < ihi))


def _loss_body(*refs):
    ins = refs[:24]
    out_cls, out_bbox, out_cent = refs[24:]
    (t_cls, t_bbox, t_ang, t_cent, s_cls, s_bbox, s_ang, s_cent) = (
        ins[0:3], ins[3:6], ins[6:9], ins[9:12],
        ins[12:15], ins[15:18], ins[18:21], ins[21:24])

    acc_cls = jnp.float32(0.0)
    acc_bbox = jnp.float32(0.0)
    acc_cent = jnp.float32(0.0)

    for img in range(_B):
        n_pos = jnp.int32(0)
        wm_sum = jnp.float32(0.0)
        mv_sum = jnp.float32(0.0)
        pos_num = jnp.float32(0.0)
        neg_num = jnp.float32(0.0)
        bb_acc = jnp.float32(0.0)
        ce_acc = jnp.float32(0.0)

        for lvl in range(3):
            tcl = t_cls[lvl][img]
            tcs = jax.nn.sigmoid(t_cent[lvl][img, 0])
            tp = jax.nn.sigmoid(tcl)
            if lvl < 2:
                mv = jnp.max(tp * tcs[None], axis=0)
            else:
                mv = jnp.max(tp, axis=0)

            ssig = jax.nn.sigmoid(s_cls[lvl][img])
            lp = jnp.sum(_bce(ssig, tp) * (tp - ssig) ** 2, axis=0)
            ln = jnp.sum(_bce(ssig, jnp.zeros_like(ssig)) * ssig ** 2, axis=0)

            bb = jnp.sum(_smooth_l1(s_bbox[lvl][img], t_bbox[lvl][img]), axis=0)
            bb = bb + _smooth_l1(s_ang[lvl][img, 0], t_ang[lvl][img, 0])
            ce = _bce(jax.nn.sigmoid(s_cent[lvl][img, 0]), tcs)

            if lvl < 2:
                coarse = _topk_mask(mv, _K)
            else:
                coarse = jnp.ones(mv.shape, dtype=bool)
            m = coarse & (mv > 0.02)

            mf = m
            n_pos = n_pos + jnp.sum(m.astype(jnp.int32))
            wm_sum = wm_sum + jnp.sum(jnp.where(mf, mv, 0.0))
            mv_sum = mv_sum + jnp.sum(mv)
            pos_num = pos_num + jnp.sum(jnp.where(mf, lp, ln))
            neg_num = neg_num + jnp.sum(jnp.where(mv > 0.0, lp, ln))
            bb_acc = bb_acc + jnp.sum(jnp.where(mf, mv * bb, 0.0))
            ce_acc = ce_acc + jnp.sum(jnp.where(mf, mv * ce, 0.0))

        has_pos = n_pos > 0
        npf = jnp.maximum(n_pos, 1).astype(jnp.float32)
        loss_cls = jnp.where(
            has_pos,
            pos_num / jnp.where(has_pos, wm_sum, 1.0),
            neg_num / mv_sum)
        loss_bbox = jnp.where(has_pos, bb_acc / (npf * 5.0) * 10.0, 0.0)
        loss_cent = jnp.where(has_pos, ce_acc / npf * 10.0, 0.0)

        acc_cls = acc_cls + loss_cls
        acc_bbox = acc_bbox + loss_bbox
        acc_cent = acc_cent + loss_cent

    out_cls[:, :] = (acc_cls / _B).reshape(1, 1)
    out_bbox[:, :] = (acc_bbox / _B).reshape(1, 1)
    out_cent[:, :] = (acc_cent / _B).reshape(1, 1)


def kernel(t_cls_0, t_cls_1, t_cls_2, t_bbox_0, t_bbox_1, t_bbox_2,
           t_ang_0, t_ang_1, t_ang_2, t_cent_0, t_cent_1, t_cent_2,
           s_cls_0, s_cls_1, s_cls_2, s_bbox_0, s_bbox_1, s_bbox_2,
           s_ang_0, s_ang_1, s_ang_2, s_cent_0, s_cent_1, s_cent_2):
    args = (t_cls_0, t_cls_1, t_cls_2, t_bbox_0, t_bbox_1, t_bbox_2,
            t_ang_0, t_ang_1, t_ang_2, t_cent_0, t_cent_1, t_cent_2,
            s_cls_0, s_cls_1, s_cls_2, s_bbox_0, s_bbox_1, s_bbox_2,
            s_ang_0, s_ang_1, s_ang_2, s_cent_0, s_cent_1, s_cent_2)
    outs = pl.pallas_call(
        _loss_body,
        out_shape=[jax.ShapeDtypeStruct((1, 1), jnp.float32)] * 3,
    )(*args)
    return (outs[0][0, 0], outs[1][0, 0], outs[2][0, 0])
